# baseline (speedup 1.0000x reference)
import functools

import jax
import jax.numpy as jnp
from jax import lax
from jax.experimental import pallas as pl
from jax.experimental.pallas import tpu as pltpu
from jax.experimental.pallas import tpu_sc as plsc

_NC = 2
_NS = 16
_C = 128


@functools.partial(jax.jit, static_argnums=(0, 1, 2, 3))
def _sc_agg_parts(n, d, steps, n_extra, x, edges):
    nw = _NC * _NS
    rpt = (n // _NS) // 8 * 8
    rem = n - rpt * _NS
    full = rpt // _C
    part = rpt - full * _C

    mesh = plsc.VectorSubcoreMesh(core_axis_name="c", subcore_axis_name="s")

    @functools.partial(
        pl.kernel,
        mesh=mesh,
        out_type=[jax.ShapeDtypeStruct((n, d), jnp.float32),
                  jax.ShapeDtypeStruct((n, d), jnp.float32)],
        scratch_types=[
            [pltpu.VMEM((2, 2 * _C), jnp.int32)] * 4,
            [pltpu.VMEM((_C, d), jnp.float32)] * 2,
            [pltpu.SemaphoreType.DMA] * 4,
            [pltpu.SemaphoreType.DMA] * 2,
            pltpu.VMEM_SHARED((n, d), jnp.float32),
        ],
    )
    def k(x_hbm, edges_hbm, out0_hbm, out1_hbm, eb, rows, si, ss,
          accum):
        cid = lax.axis_index("c")
        sid = lax.axis_index("s")
        wid = cid * _NS + sid
        ebase = wid * steps * _C

        def pidx_start(p, b):
            off = ebase + p * 2 * _C
            pltpu.async_copy(edges_hbm.at[:, pl.ds(off, 2 * _C)], eb[b],
                             si[b])

        def pidx_wait(p, b):
            off = ebase + p * 2 * _C
            pltpu.make_async_copy(edges_hbm.at[:, pl.ds(off, 2 * _C)],
                                  eb[b], si[b]).wait()

        def gather_start(b, h, r_):
            pltpu.async_copy(x_hbm.at[eb[b].at[0, pl.ds(h * _C, _C)]],
                             rows[r_], ss[r_])

        def gather_wait(b, h, r_):
            pltpu.make_async_copy(x_hbm.at[eb[b].at[0, pl.ds(h * _C, _C)]],
                                  rows[r_], ss[r_]).wait()

        def scat_start(b, h, r_):
            pltpu.async_copy(rows[r_],
                             accum.at[eb[b].at[1, pl.ds(h * _C, _C)]],
                             ss[r_], add=True)

        def scat_wait(b, h, r_):
            pltpu.make_async_copy(rows[r_],
                                  accum.at[eb[b].at[1, pl.ds(h * _C, _C)]],
                                  ss[r_]).wait()

        for p_ in range(4):
            pidx_start(p_, p_)
        pidx_wait(0, 0)
        gather_start(0, 0, 0)

        zv = jnp.zeros((16,), jnp.float32)

        def zrow(r, _):
            for cc in range(d // 16):
                rows[1][r, pl.ds(cc * 16, 16)] = zv
            return 0

        lax.fori_loop(0, _C, zrow, 0)
        zbase = sid * rpt
        for b in range(full):
            pltpu.sync_copy(rows[1], accum.at[pl.ds(zbase + b * _C, _C)])
        if part:
            pltpu.sync_copy(rows[1].at[pl.ds(0, part)],
                            accum.at[pl.ds(zbase + full * _C, part)])
        if rem:
            @pl.when(sid == 0)
            def _():
                pltpu.sync_copy(rows[1].at[pl.ds(0, rem)],
                                accum.at[pl.ds(rpt * _NS, rem)])

        plsc.subcore_barrier()

        def slot(q, s8, do_scat_wait=True, do_refill=True, do_next=True):
            b_cur = (s8 // 2) % 4
            h_cur = s8 % 2
            b_nxt = ((s8 + 1) // 2) % 4
            h_nxt = (s8 + 1) % 2
            if do_next and h_nxt == 0:
                pidx_wait((q + 1) // 2, b_nxt)
            if do_scat_wait:
                scat_wait(((s8 - 1) // 2) % 4, (s8 - 1) % 2, (s8 + 1) % 2)
            if do_next:
                gather_start(b_nxt, h_nxt, (s8 + 1) % 2)
            if do_refill and s8 % 2 == 0:
                pidx_start(q // 2 + 3, (s8 // 2 + 3) % 4)
            gather_wait(b_cur, h_cur, s8 % 2)
            scat_start(b_cur, h_cur, s8 % 2)

        slot(0, 0, do_scat_wait=False, do_refill=False)
        for q in range(1, 8):
            slot(q, q)

        def oct(i, _):
            c = 8 * i
            for s8 in range(8):
                slot(c + s8, s8)
            return 0

        lax.fori_loop(1, (steps - 14) // 8 + 1, oct, 0)

        for q in range(steps - 6, steps):
            s8 = q % 8
            slot(q, s8,
                 do_refill=(q // 2 + 3 < steps // 2),
                 do_next=(q + 1 < steps))
        scat_wait(((steps - 1) % 8 // 2) % 4, (steps - 1) % 2,
                  (steps - 1) % 2)

        if n_extra:
            @pl.when(jnp.logical_and(wid % 8 == 0, wid // 8 < n_extra))
            def _():
                off = nw * steps * _C + (wid // 8) * _C
                pltpu.sync_copy(edges_hbm.at[:, pl.ds(off, _C)],
                                eb[0].at[:, pl.ds(0, _C)])
                sidx = eb[0].at[0, pl.ds(0, _C)]
                pltpu.async_copy(x_hbm.at[sidx], rows[0], ss[0])
                pltpu.make_async_copy(x_hbm.at[sidx], rows[0],
                                      ss[0]).wait()
                pltpu.sync_copy(rows[0],
                                accum.at[eb[0].at[1, pl.ds(0, _C)]],
                                add=True)

        plsc.subcore_barrier()

        @pl.when(cid == 0)
        def _():
            pltpu.sync_copy(accum.at[pl.ds(sid * rpt, rpt)],
                            out0_hbm.at[pl.ds(sid * rpt, rpt)])
            if rem:
                @pl.when(sid == 0)
                def _():
                    pltpu.sync_copy(accum.at[pl.ds(rpt * _NS, rem)],
                                    out0_hbm.at[pl.ds(rpt * _NS, rem)])

        @pl.when(cid == 1)
        def _():
            pltpu.sync_copy(accum.at[pl.ds(sid * rpt, rpt)],
                            out1_hbm.at[pl.ds(sid * rpt, rpt)])
            if rem:
                @pl.when(sid == 0)
                def _():
                    pltpu.sync_copy(accum.at[pl.ds(rpt * _NS, rem)],
                                    out1_hbm.at[pl.ds(rpt * _NS, rem)])

    return k(x, edges)


_DNUM = (((1,), (1,)), ((), ()))


def _tc_final_body(p0_ref, p1_ref, x_ref, w1l_ref, b1_ref, w1r_ref,
                   w2l_ref, b2_ref, w2r_ref, loc_ref, scale_ref):
    agg = p0_ref[...] + p1_ref[...]
    xb = x_ref[...]
    h1 = (lax.dot_general(agg, w1l_ref[...], _DNUM,
                          preferred_element_type=jnp.float32)
          + lax.dot_general(xb, w1r_ref[...], _DNUM,
                            preferred_element_type=jnp.float32)
          + b1_ref[...])
    loc_ref[...] = jnp.clip(h1, -100.0, 100.0)
    h2 = (lax.dot_general(agg, w2l_ref[...], _DNUM,
                          preferred_element_type=jnp.float32)
          + lax.dot_general(xb, w2r_ref[...], _DNUM,
                            preferred_element_type=jnp.float32)
          + b2_ref[...])
    sp = jnp.maximum(h2, 0.0) + jnp.log1p(jnp.exp(-jnp.abs(h2)))
    scale_ref[...] = jnp.minimum(sp + 0.001, 100.0)


def _tc_final(p0, p1, x, w1l, b1, w1r, w2l, b2, w2r):
    n, d = p0.shape
    blk = 2000
    row_spec = pl.BlockSpec((blk, d), lambda i: (i, 0))
    w_spec = pl.BlockSpec((d, d), lambda i: (0, 0))
    b_spec = pl.BlockSpec((1, d), lambda i: (0, 0))
    return pl.pallas_call(
        _tc_final_body,
        grid=(n // blk,),
        in_specs=[row_spec, row_spec, row_spec,
                  w_spec, b_spec, w_spec, w_spec, b_spec, w_spec],
        out_specs=[row_spec, row_spec],
        out_shape=[jax.ShapeDtypeStruct((n, d), jnp.float32),
                   jax.ShapeDtypeStruct((n, d), jnp.float32)],
    )(p0, p1, x, w1l, b1, w1r, w2l, b2, w2r)


def kernel(x, edge_index, W1l, b1l, W1r, W2l, b2l, W2r):
    n, d = x.shape
    e = edge_index.shape[1]
    nw = _NC * _NS
    steps = e // (_C * nw)
    steps = steps // 2 * 2
    n_extra = (e - nw * steps * _C) // _C
    if not (e % _C == 0 and n_extra <= 4 and steps % 8 == 6
            and steps >= 14):
        raise NotImplementedError("edge count layout not supported")
    p0, p1 = _sc_agg_parts(n, d, steps, n_extra, x, edge_index)
    loc, scale = _tc_final(p0, p1, x, W1l, b1l.reshape(1, d), W1r,
                           W2l, b2l.reshape(1, d), W2r)
    return (loc, scale)


# speedup vs baseline: 1.0088x; 1.0088x over previous
import functools

import jax
import jax.numpy as jnp
from jax import lax
from jax.experimental import pallas as pl
from jax.experimental.pallas import tpu as pltpu
from jax.experimental.pallas import tpu_sc as plsc

_NC = 2
_NS = 16
_C = 128


@functools.partial(jax.jit, static_argnums=(0, 1, 2, 3))
def _sc_agg_parts(n, d, steps, n_extra, x, edges):
    nw = _NC * _NS
    rpt = (n // _NS) // 8 * 8
    rem = n - rpt * _NS
    full = rpt // _C
    part = rpt - full * _C

    mesh = plsc.VectorSubcoreMesh(core_axis_name="c", subcore_axis_name="s")

    @functools.partial(
        pl.kernel,
        mesh=mesh,
        out_type=[jax.ShapeDtypeStruct((n, d), jnp.float32),
                  jax.ShapeDtypeStruct((n, d), jnp.float32)],
        scratch_types=[
            [pltpu.VMEM((2, _C), jnp.int32)] * 4,
            [pltpu.VMEM((_C, d), jnp.float32)] * 2,
            [pltpu.SemaphoreType.DMA] * 4,
            [pltpu.SemaphoreType.DMA] * 2,
            pltpu.VMEM_SHARED((n, d), jnp.float32),
        ],
    )
    def k(x_hbm, edges_hbm, out0_hbm, out1_hbm, eb, rows, si, ss,
          accum):
        cid = lax.axis_index("c")
        sid = lax.axis_index("s")
        wid = cid * _NS + sid
        ebase = wid * steps * _C

        def idx_start(c, k_):
            off = ebase + c * _C
            pltpu.async_copy(edges_hbm.at[:, pl.ds(off, _C)], eb[k_],
                             si[k_])

        def idx_wait(c, k_):
            off = ebase + c * _C
            pltpu.make_async_copy(edges_hbm.at[:, pl.ds(off, _C)], eb[k_],
                                  si[k_]).wait()

        def gather_start(k_, r_):
            pltpu.async_copy(x_hbm.at[eb[k_].at[0]], rows[r_], ss[r_])

        def gather_wait(k_, r_):
            pltpu.make_async_copy(x_hbm.at[eb[k_].at[0]], rows[r_],
                                  ss[r_]).wait()

        def scat_start(k_, r_):
            pltpu.async_copy(rows[r_], accum.at[eb[k_].at[1]], ss[r_],
                             add=True)

        def scat_wait(k_, r_):
            pltpu.make_async_copy(rows[r_], accum.at[eb[k_].at[1]],
                                  ss[r_]).wait()

        for k_ in range(4):
            idx_start(k_, k_)
        idx_wait(0, 0)
        gather_start(0, 0)

        zv = jnp.zeros((16,), jnp.float32)

        def zrow(r, _):
            for cc in range(d // 16):
                rows[1][r, pl.ds(cc * 16, 16)] = zv
            return 0

        lax.fori_loop(0, _C, zrow, 0)
        zbase = sid * rpt
        for b in range(full):
            pltpu.sync_copy(rows[1], accum.at[pl.ds(zbase + b * _C, _C)])
        if part:
            pltpu.sync_copy(rows[1].at[pl.ds(0, part)],
                            accum.at[pl.ds(zbase + full * _C, part)])
        if rem:
            @pl.when(sid == 0)
            def _():
                pltpu.sync_copy(rows[1].at[pl.ds(0, rem)],
                                accum.at[pl.ds(rpt * _NS, rem)])

        plsc.subcore_barrier()

        def slot(q, k_, do_scat_wait=True, do_refill=True, do_next=True):
            if do_next:
                idx_wait(q + 1, (k_ + 1) % 4)
            if do_scat_wait:
                scat_wait((k_ - 1) % 4, (k_ + 1) % 2)
            if do_next:
                gather_start((k_ + 1) % 4, (k_ + 1) % 2)
            if do_refill:
                idx_start(q + 3, (k_ - 1) % 4)
            gather_wait(k_ % 4, k_ % 2)
            scat_start(k_ % 4, k_ % 2)

        slot(0, 0, do_scat_wait=False, do_refill=False)
        slot(1, 1)
        slot(2, 2)
        slot(3, 3)

        def quad(j, _):
            c = 4 * j
            for k_ in range(4):
                slot(c + k_, k_)
            return 0

        lax.fori_loop(1, (steps - 6) // 4, quad, 0)

        for q in range(steps - 6, steps):
            k_ = q % 4
            slot(q, k_,
                 do_refill=(q + 3 < steps),
                 do_next=(q + 1 < steps))
        scat_wait((steps - 1) % 4, (steps - 1) % 2)

        if n_extra:
            @pl.when(jnp.logical_and(wid % 8 == 0, wid // 8 < n_extra))
            def _():
                off = nw * steps * _C + (wid // 8) * _C
                pltpu.sync_copy(edges_hbm.at[:, pl.ds(off, _C)], eb[0])
                pltpu.async_copy(x_hbm.at[eb[0].at[0]], rows[0], ss[0])
                pltpu.make_async_copy(x_hbm.at[eb[0].at[0]], rows[0],
                                      ss[0]).wait()
                pltpu.sync_copy(rows[0], accum.at[eb[0].at[1]], add=True)

        plsc.subcore_barrier()

        @pl.when(cid == 0)
        def _():
            pltpu.sync_copy(accum.at[pl.ds(sid * rpt, rpt)],
                            out0_hbm.at[pl.ds(sid * rpt, rpt)])
            if rem:
                @pl.when(sid == 0)
                def _():
                    pltpu.sync_copy(accum.at[pl.ds(rpt * _NS, rem)],
                                    out0_hbm.at[pl.ds(rpt * _NS, rem)])

        @pl.when(cid == 1)
        def _():
            pltpu.sync_copy(accum.at[pl.ds(sid * rpt, rpt)],
                            out1_hbm.at[pl.ds(sid * rpt, rpt)])
            if rem:
                @pl.when(sid == 0)
                def _():
                    pltpu.sync_copy(accum.at[pl.ds(rpt * _NS, rem)],
                                    out1_hbm.at[pl.ds(rpt * _NS, rem)])

    return k(x, edges)


_DNUM = (((1,), (1,)), ((), ()))


def _tc_final_body(p0_ref, p1_ref, x_ref, w1l_ref, b1_ref, w1r_ref,
                   w2l_ref, b2_ref, w2r_ref, loc_ref, scale_ref):
    agg = p0_ref[...] + p1_ref[...]
    xb = x_ref[...]
    h1 = (lax.dot_general(agg, w1l_ref[...], _DNUM,
                          preferred_element_type=jnp.float32)
          + lax.dot_general(xb, w1r_ref[...], _DNUM,
                            preferred_element_type=jnp.float32)
          + b1_ref[...])
    loc_ref[...] = jnp.clip(h1, -100.0, 100.0)
    h2 = (lax.dot_general(agg, w2l_ref[...], _DNUM,
                          preferred_element_type=jnp.float32)
          + lax.dot_general(xb, w2r_ref[...], _DNUM,
                            preferred_element_type=jnp.float32)
          + b2_ref[...])
    sp = jnp.maximum(h2, 0.0) + jnp.log1p(jnp.exp(-jnp.abs(h2)))
    scale_ref[...] = jnp.minimum(sp + 0.001, 100.0)


def _tc_final(p0, p1, x, w1l, b1, w1r, w2l, b2, w2r):
    n, d = p0.shape
    blk = 2000
    row_spec = pl.BlockSpec((blk, d), lambda i: (i, 0))
    w_spec = pl.BlockSpec((d, d), lambda i: (0, 0))
    b_spec = pl.BlockSpec((1, d), lambda i: (0, 0))
    return pl.pallas_call(
        _tc_final_body,
        grid=(n // blk,),
        in_specs=[row_spec, row_spec, row_spec,
                  w_spec, b_spec, w_spec, w_spec, b_spec, w_spec],
        out_specs=[row_spec, row_spec],
        out_shape=[jax.ShapeDtypeStruct((n, d), jnp.float32),
                   jax.ShapeDtypeStruct((n, d), jnp.float32)],
    )(p0, p1, x, w1l, b1, w1r, w2l, b2, w2r)


def kernel(x, edge_index, W1l, b1l, W1r, W2l, b2l, W2r):
    n, d = x.shape
    e = edge_index.shape[1]
    nw = _NC * _NS
    steps = e // (_C * nw)
    steps = steps // 2 * 2
    n_extra = (e - nw * steps * _C) // _C
    if not (e % _C == 0 and n_extra <= 4 and steps % 4 == 2 and steps >= 8):
        raise NotImplementedError("edge count layout not supported")
    p0, p1 = _sc_agg_parts(n, d, steps, n_extra, x, edge_index)
    loc, scale = _tc_final(p0, p1, x, W1l, b1l.reshape(1, d), W1r,
                           W2l, b2l.reshape(1, d), W2r)
    return (loc, scale)
